# baseline (speedup 1.0000x reference)
import functools

import jax
import jax.numpy as jnp
from jax import lax
from jax.experimental import pallas as pl
from jax.experimental.pallas import tpu as pltpu
from jax.experimental.pallas import tpu_sc as plsc

_NC = 2
_NS = 16
_NW = _NC * _NS
_CH = 128

_N = 10000
_E = 320000
_H = 64
_NPAD = 10240
_RPT = _NPAD // _NS
_NBUF = 4
_CPT = 80
_RPN = _N // _NS
_EPAD = _NW * _CPT * _CH

def _make_mesh():
    return plsc.VectorSubcoreMesh(
        core_axis_name="c", subcore_axis_name="s", num_cores=_NC, num_subcores=_NS
    )


def _sc_deg_body(dst_hbm, ew_hbm, z_hbm, out_hbm, dst_v, ew_v, acc):
    c = lax.axis_index("c")
    s = lax.axis_index("s")
    gid = c * _NS + s
    pltpu.sync_copy(z_hbm.at[pl.ds(0, _RPT)], acc.at[pl.ds(s * _RPT, _RPT)])
    plsc.subcore_barrier()
    pltpu.sync_copy(dst_hbm.at[gid], dst_v)
    pltpu.sync_copy(ew_hbm.at[gid], ew_v)

    def body(j, carry):
        pltpu.sync_copy(ew_v.at[j], acc.at[dst_v.at[j]], add=True)
        return carry

    lax.fori_loop(0, _CPT, body, 0)
    plsc.subcore_barrier()
    pltpu.sync_copy(acc.at[pl.ds(s * _RPT, _RPT)], out_hbm.at[c, pl.ds(s * _RPT, _RPT)])


@functools.cache
def _sc_deg():
    return pl.kernel(
        _sc_deg_body,
        out_type=jax.ShapeDtypeStruct((_NC, _NPAD), jnp.float32),
        mesh=_make_mesh(),
        scratch_types=[
            pltpu.VMEM((_CPT, _CH), jnp.int32),
            pltpu.VMEM((_CPT, _CH), jnp.float32),
            pltpu.VMEM_SHARED((_NPAD,), jnp.float32),
        ],
    )


def _sc_agg_body(hs_hbm, src_hbm, dst_hbm, ew_hbm, z_hbm, out_hbm,
                 src_v, dst_v, ew_v, rbf0, rbf1, rbf2, rbf3, rf0, rf1, hs_s, acc,
                 gs0, gs1, gs2, gs3, ss0, ss1):
    rbf = (rbf0, rbf1, rbf2, rbf3)
    rf = (rf0, rf1)
    gsems = (gs0, gs1, gs2, gs3)
    ssems = (ss0, ss1)
    c = lax.axis_index("c")
    s = lax.axis_index("s")
    gid = c * _NS + s
    for k in range(_RPT // _CH):
        pltpu.sync_copy(z_hbm, acc.at[pl.ds(s * _RPT + k * _CH, _CH)])
    plsc.subcore_barrier()
    pltpu.sync_copy(src_hbm.at[gid], src_v)
    pltpu.sync_copy(dst_hbm.at[gid], dst_v)
    pltpu.sync_copy(ew_hbm.at[gid], ew_v)

    pltpu.sync_copy(hs_hbm.at[pl.ds(s * _RPN, _RPN)], hs_s.at[pl.ds(s * _RPN, _RPN)])
    plsc.subcore_barrier()

    lanes = lax.iota(jnp.int32, 16)
    col_even = (lanes * 2, lanes * 2 + 32)
    col_odd = (lanes * 2 + 1, lanes * 2 + 33)
    mask_hi = jnp.full((16,), -65536, jnp.int32)

    def _scale(bin_, bout, j):
        def grp(g, carry):
            ewg = ew_v[j, pl.ds(g * 16, 16)]
            for k in range(16):
                e = g * 16 + k
                wv = jnp.full((16,), ewg[k], jnp.float32)
                ev = jnp.full((16,), e, jnp.int32)
                for q in range(2):
                    x = plsc.bitcast(bin_[e, pl.ds(q * 32, 32)], jnp.int32)
                    lo = plsc.bitcast(x << 16, jnp.float32)
                    hi = plsc.bitcast(x & mask_hi, jnp.float32)
                    plsc.store_scatter(bout, [ev, col_even[q]], lo * wv)
                    plsc.store_scatter(bout, [ev, col_odd[q]], hi * wv)
            return carry

        lax.fori_loop(0, _CH // 16, grp, 0)

    pltpu.async_copy(hs_s.at[src_v.at[0]], rbf[0], gsems[0])
    pltpu.async_copy(hs_s.at[src_v.at[1]], rbf[1], gsems[1])

    def body(p, carry):
        for b in range(_NBUF):
            j = p * _NBUF + b
            bn = (b + 2) % _NBUF
            f = b % 2
            pltpu.make_async_copy(hs_s.at[src_v.at[j]], rbf[b], gsems[b]).wait()

            @pl.when(j >= 2)
            def _():
                pltpu.make_async_copy(
                    rf[f], acc.at[dst_v.at[j - 2]], ssems[f]
                ).wait()

            _scale(rbf[b], rf[f], j)
            pltpu.async_copy(rf[f], acc.at[dst_v.at[j]], ssems[f], add=True)

            @pl.when(j + 2 < _CPT)
            def _():
                pltpu.async_copy(hs_s.at[src_v.at[j + 2]], rbf[bn], gsems[bn])
        return carry

    lax.fori_loop(0, _CPT // _NBUF, body, 0)
    for t in (_CPT - 2, _CPT - 1):
        pltpu.make_async_copy(rf[t % 2], acc.at[dst_v.at[t]], ssems[t % 2]).wait()
    plsc.subcore_barrier()
    pltpu.sync_copy(acc.at[pl.ds(s * _RPT, _RPT)], out_hbm.at[c, pl.ds(s * _RPT, _RPT)])


@functools.cache
def _sc_agg():
    return pl.kernel(
        _sc_agg_body,
        out_type=jax.ShapeDtypeStruct((_NC, _NPAD, _H), jnp.float32),
        mesh=_make_mesh(),
        scratch_types=[
            pltpu.VMEM((_CPT, _CH), jnp.int32),
            pltpu.VMEM((_CPT, _CH), jnp.int32),
            pltpu.VMEM((_CPT, _CH), jnp.float32),
            pltpu.VMEM((_CH, _H), jnp.bfloat16),
            pltpu.VMEM((_CH, _H), jnp.bfloat16),
            pltpu.VMEM((_CH, _H), jnp.bfloat16),
            pltpu.VMEM((_CH, _H), jnp.bfloat16),
            pltpu.VMEM((_CH, _H), jnp.float32),
            pltpu.VMEM((_CH, _H), jnp.float32),
            pltpu.VMEM_SHARED((_N, _H), jnp.bfloat16),
            pltpu.VMEM_SHARED((_NPAD, _H), jnp.float32),
            pltpu.SemaphoreType.DMA,
            pltpu.SemaphoreType.DMA,
            pltpu.SemaphoreType.DMA,
            pltpu.SemaphoreType.DMA,
            pltpu.SemaphoreType.DMA,
            pltpu.SemaphoreType.DMA,
        ],
        compiler_params=pltpu.CompilerParams(
            use_tc_tiling_on_sc=False, needs_layout_passes=False
        ),
    )


_BR = 1000


def _tc_b_body(x_ref, w0_ref, degt_ref, h_ref, hs_ref):
    d = 1.0 + degt_ref[:, 0:1] + degt_ref[:, 1:2]
    dis = lax.rsqrt(d)
    h = jnp.dot(x_ref[...], w0_ref[...], preferred_element_type=jnp.float32)
    h_ref[...] = h
    hs_ref[...] = (h * dis).astype(jnp.bfloat16)


def _tc_stage_b(x, w0, degt):
    n = x.shape[0]
    grid = n // _BR
    return pl.pallas_call(
        _tc_b_body,
        grid=(grid,),
        in_specs=[
            pl.BlockSpec((_BR, x.shape[1]), lambda i: (i, 0)),
            pl.BlockSpec(w0.shape, lambda i: (0, 0)),
            pl.BlockSpec((_BR, 2), lambda i: (i, 0)),
        ],
        out_specs=[
            pl.BlockSpec((_BR, _H), lambda i: (i, 0)),
            pl.BlockSpec((_BR, _H), lambda i: (i, 0)),
        ],
        out_shape=[
            jax.ShapeDtypeStruct((n, _H), jnp.float32),
            jax.ShapeDtypeStruct((n, _H), jnp.bfloat16),
        ],
    )(x, w0, degt)


def _elu(t):
    return jnp.where(t > 0.0, t, jnp.exp(jnp.minimum(t, 0.0)) - 1.0)


def _tc_d_body(aggp_ref, h0_ref, degt_ref, b_ref, w_ref, h_ref, hs_ref):
    d = 1.0 + degt_ref[:, 0:1] + degt_ref[:, 1:2]
    dis = lax.rsqrt(d)
    agg = aggp_ref[0] + aggp_ref[1]
    t = dis * agg + h0_ref[...] / d + b_ref[...]
    e = _elu(t)
    h = jnp.dot(e, w_ref[...], preferred_element_type=jnp.float32)
    h_ref[...] = h
    hs_ref[...] = (h * dis).astype(jnp.bfloat16)


def _tc_stage_d(aggp, h0, degt, b, w):
    n = h0.shape[0]
    grid = n // _BR
    return pl.pallas_call(
        _tc_d_body,
        grid=(grid,),
        in_specs=[
            pl.BlockSpec((_NC, _BR, _H), lambda i: (0, i, 0)),
            pl.BlockSpec((_BR, _H), lambda i: (i, 0)),
            pl.BlockSpec((_BR, 2), lambda i: (i, 0)),
            pl.BlockSpec(b.shape, lambda i: (0, 0)),
            pl.BlockSpec(w.shape, lambda i: (0, 0)),
        ],
        out_specs=[
            pl.BlockSpec((_BR, _H), lambda i: (i, 0)),
            pl.BlockSpec((_BR, _H), lambda i: (i, 0)),
        ],
        out_shape=[
            jax.ShapeDtypeStruct((n, _H), jnp.float32),
            jax.ShapeDtypeStruct((n, _H), jnp.bfloat16),
        ],
    )(aggp, h0, degt, b, w)


def _tc_f_body(aggp_ref, h1_ref, degt_ref, b_ref, wl_ref, bl_ref, out_ref):
    d = 1.0 + degt_ref[:, 0:1] + degt_ref[:, 1:2]
    dis = lax.rsqrt(d)
    agg = aggp_ref[0] + aggp_ref[1]
    t = dis * agg + h1_ref[...] / d + b_ref[...]
    e = _elu(t)
    logits = jnp.dot(e, wl_ref[...], preferred_element_type=jnp.float32) + bl_ref[...]
    m = jnp.max(logits, axis=1, keepdims=True)
    shifted = logits - m
    lse = jnp.log(jnp.sum(jnp.exp(shifted), axis=1, keepdims=True))
    out_ref[...] = shifted - lse


def _tc_stage_f(aggp, h1, degt, b, wl, bl):
    n = h1.shape[0]
    c = wl.shape[1]
    grid = n // _BR
    return pl.pallas_call(
        _tc_f_body,
        grid=(grid,),
        in_specs=[
            pl.BlockSpec((_NC, _BR, _H), lambda i: (0, i, 0)),
            pl.BlockSpec((_BR, _H), lambda i: (i, 0)),
            pl.BlockSpec((_BR, 2), lambda i: (i, 0)),
            pl.BlockSpec(b.shape, lambda i: (0, 0)),
            pl.BlockSpec(wl.shape, lambda i: (0, 0)),
            pl.BlockSpec(bl.shape, lambda i: (0, 0)),
        ],
        out_specs=pl.BlockSpec((_BR, c), lambda i: (i, 0)),
        out_shape=jax.ShapeDtypeStruct((n, c), jnp.float32),
    )(aggp, h1, degt, b, wl, bl)


@jax.jit
def kernel(x, edge_index, edge_attr, W0, b0, W1, b1, Wl, bl):
    src = edge_index[0]
    dst = edge_index[1]
    ew = edge_attr[:, 0]

    pad = _EPAD - _E
    src3 = jnp.pad(src, (0, pad)).reshape(_NW, _CPT, _CH)
    dst3 = jnp.pad(dst, (0, pad)).reshape(_NW, _CPT, _CH)
    ew3 = jnp.pad(ew, (0, pad)).reshape(_NW, _CPT, _CH)

    zrow = jnp.zeros((_NPAD,), jnp.float32)
    zblk = jnp.zeros((_CH, _H), jnp.float32)

    degp = _sc_deg()(dst3, ew3, zrow)
    degt = degp.T

    h0, hs0 = _tc_stage_b(x, W0, degt)
    agg1 = _sc_agg()(hs0, src3, dst3, ew3, zblk)
    h1, hs1 = _tc_stage_d(agg1, h0, degt, b0.reshape(1, _H), W1)
    agg2 = _sc_agg()(hs1, src3, dst3, ew3, zblk)
    out = _tc_stage_f(agg2, h1, degt, b1.reshape(1, _H), Wl, bl.reshape(1, -1))
    return out


# speedup vs baseline: 1.2817x; 1.2817x over previous
import functools

import jax
import jax.numpy as jnp
from jax import lax
from jax.experimental import pallas as pl
from jax.experimental.pallas import tpu as pltpu
from jax.experimental.pallas import tpu_sc as plsc

_NC = 2
_NS = 16
_NW = _NC * _NS
_CH = 128

_N = 10000
_E = 320000
_H = 64
_NPAD = 10240
_RPT = _NPAD // _NS
_NBUF = 2
_CPT = 80
_RPN = _N // _NS
_EPAD = _NW * _CPT * _CH

def _make_mesh():
    return plsc.VectorSubcoreMesh(
        core_axis_name="c", subcore_axis_name="s", num_cores=_NC, num_subcores=_NS
    )


def _sc_deg_body(dst_hbm, ew_hbm, z_hbm, out_hbm, dst_v, ew_v, acc):
    c = lax.axis_index("c")
    s = lax.axis_index("s")
    gid = c * _NS + s
    pltpu.sync_copy(z_hbm.at[pl.ds(0, _RPT)], acc.at[pl.ds(s * _RPT, _RPT)])
    plsc.subcore_barrier()
    pltpu.sync_copy(dst_hbm.at[gid], dst_v)
    pltpu.sync_copy(ew_hbm.at[gid], ew_v)

    def body(j, carry):
        pltpu.sync_copy(ew_v.at[j], acc.at[dst_v.at[j]], add=True)
        return carry

    lax.fori_loop(0, _CPT, body, 0)
    plsc.subcore_barrier()
    pltpu.sync_copy(acc.at[pl.ds(s * _RPT, _RPT)], out_hbm.at[c, pl.ds(s * _RPT, _RPT)])


@functools.cache
def _sc_deg():
    return pl.kernel(
        _sc_deg_body,
        out_type=jax.ShapeDtypeStruct((_NC, _NPAD), jnp.float32),
        mesh=_make_mesh(),
        scratch_types=[
            pltpu.VMEM((_CPT, _CH), jnp.int32),
            pltpu.VMEM((_CPT, _CH), jnp.float32),
            pltpu.VMEM_SHARED((_NPAD,), jnp.float32),
        ],
    )


def _sc_agg_body(hs_hbm, src_hbm, dst_hbm, ew_hbm, z_hbm, out_hbm,
                 src_v, dst_v, ew_v, rows0, rows1, hs_s, acc,
                 gs0, ss0, ss1):
    rows = (rows0, rows1)
    ssems = (ss0, ss1)
    c = lax.axis_index("c")
    s = lax.axis_index("s")
    gid = c * _NS + s
    for k in range(_RPT // _CH):
        pltpu.sync_copy(z_hbm, acc.at[pl.ds(s * _RPT + k * _CH, _CH)])
    plsc.subcore_barrier()
    pltpu.sync_copy(src_hbm.at[gid], src_v)
    pltpu.sync_copy(dst_hbm.at[gid], dst_v)
    pltpu.sync_copy(ew_hbm.at[gid], ew_v)

    def _scale(buf, j):
        for g in range(_CH // 16):
            ewg = ew_v[j, pl.ds(g * 16, 16)]
            for k in range(16):
                e = g * 16 + k
                wv = jnp.full((16,), ewg[k], jnp.float32)
                for q in range(_H // 16):
                    buf[e, pl.ds(q * 16, 16)] = buf[e, pl.ds(q * 16, 16)] * wv

    pltpu.sync_copy(hs_hbm.at[pl.ds(s * _RPN, _RPN)], hs_s.at[pl.ds(s * _RPN, _RPN)])
    plsc.subcore_barrier()

    def body(p, carry):
        for b in range(_NBUF):
            j = p * _NBUF + b
            bo = (b + 1) % _NBUF
            pltpu.async_copy(hs_s.at[src_v.at[j]], rows[b], gs0).wait()
            _scale(rows[b], j)

            @pl.when(j >= 1)
            def _():
                pltpu.make_async_copy(
                    rows[bo], acc.at[dst_v.at[j - 1]], ssems[bo]
                ).wait()

            pltpu.async_copy(rows[b], acc.at[dst_v.at[j]], ssems[b], add=True)
        return carry

    lax.fori_loop(0, _CPT // _NBUF, body, 0)
    bl = (_CPT - 1) % _NBUF
    pltpu.make_async_copy(rows[bl], acc.at[dst_v.at[_CPT - 1]], ssems[bl]).wait()
    plsc.subcore_barrier()
    pltpu.sync_copy(acc.at[pl.ds(s * _RPT, _RPT)], out_hbm.at[c, pl.ds(s * _RPT, _RPT)])


@functools.cache
def _sc_agg():
    return pl.kernel(
        _sc_agg_body,
        out_type=jax.ShapeDtypeStruct((_NC, _NPAD, _H), jnp.float32),
        mesh=_make_mesh(),
        scratch_types=[
            pltpu.VMEM((_CPT, _CH), jnp.int32),
            pltpu.VMEM((_CPT, _CH), jnp.int32),
            pltpu.VMEM((_CPT, _CH), jnp.float32),
            pltpu.VMEM((_CH, _H), jnp.float32),
            pltpu.VMEM((_CH, _H), jnp.float32),
            pltpu.VMEM_SHARED((_N, _H), jnp.float32),
            pltpu.VMEM_SHARED((_NPAD, _H), jnp.float32),
            pltpu.SemaphoreType.DMA,
            pltpu.SemaphoreType.DMA,
            pltpu.SemaphoreType.DMA,
        ],
        compiler_params=pltpu.CompilerParams(use_tc_tiling_on_sc=False),
    )


_BR = 1000


def _tc_b_body(x_ref, w0_ref, degt_ref, h_ref, hs_ref):
    d = 1.0 + degt_ref[:, 0:1] + degt_ref[:, 1:2]
    dis = lax.rsqrt(d)
    h = jnp.dot(x_ref[...], w0_ref[...], preferred_element_type=jnp.float32)
    h_ref[...] = h
    hs_ref[...] = h * dis


def _tc_stage_b(x, w0, degt):
    n = x.shape[0]
    grid = n // _BR
    return pl.pallas_call(
        _tc_b_body,
        grid=(grid,),
        in_specs=[
            pl.BlockSpec((_BR, x.shape[1]), lambda i: (i, 0)),
            pl.BlockSpec(w0.shape, lambda i: (0, 0)),
            pl.BlockSpec((_BR, 2), lambda i: (i, 0)),
        ],
        out_specs=[
            pl.BlockSpec((_BR, _H), lambda i: (i, 0)),
            pl.BlockSpec((_BR, _H), lambda i: (i, 0)),
        ],
        out_shape=[
            jax.ShapeDtypeStruct((n, _H), jnp.float32),
            jax.ShapeDtypeStruct((n, _H), jnp.float32),
        ],
    )(x, w0, degt)


def _elu(t):
    return jnp.where(t > 0.0, t, jnp.exp(jnp.minimum(t, 0.0)) - 1.0)


def _tc_d_body(aggp_ref, h0_ref, degt_ref, b_ref, w_ref, h_ref, hs_ref):
    d = 1.0 + degt_ref[:, 0:1] + degt_ref[:, 1:2]
    dis = lax.rsqrt(d)
    agg = aggp_ref[0] + aggp_ref[1]
    t = dis * agg + h0_ref[...] / d + b_ref[...]
    e = _elu(t)
    h = jnp.dot(e, w_ref[...], preferred_element_type=jnp.float32)
    h_ref[...] = h
    hs_ref[...] = h * dis


def _tc_stage_d(aggp, h0, degt, b, w):
    n = h0.shape[0]
    grid = n // _BR
    return pl.pallas_call(
        _tc_d_body,
        grid=(grid,),
        in_specs=[
            pl.BlockSpec((_NC, _BR, _H), lambda i: (0, i, 0)),
            pl.BlockSpec((_BR, _H), lambda i: (i, 0)),
            pl.BlockSpec((_BR, 2), lambda i: (i, 0)),
            pl.BlockSpec(b.shape, lambda i: (0, 0)),
            pl.BlockSpec(w.shape, lambda i: (0, 0)),
        ],
        out_specs=[
            pl.BlockSpec((_BR, _H), lambda i: (i, 0)),
            pl.BlockSpec((_BR, _H), lambda i: (i, 0)),
        ],
        out_shape=[
            jax.ShapeDtypeStruct((n, _H), jnp.float32),
            jax.ShapeDtypeStruct((n, _H), jnp.float32),
        ],
    )(aggp, h0, degt, b, w)


def _tc_f_body(aggp_ref, h1_ref, degt_ref, b_ref, wl_ref, bl_ref, out_ref):
    d = 1.0 + degt_ref[:, 0:1] + degt_ref[:, 1:2]
    dis = lax.rsqrt(d)
    agg = aggp_ref[0] + aggp_ref[1]
    t = dis * agg + h1_ref[...] / d + b_ref[...]
    e = _elu(t)
    logits = jnp.dot(e, wl_ref[...], preferred_element_type=jnp.float32) + bl_ref[...]
    m = jnp.max(logits, axis=1, keepdims=True)
    shifted = logits - m
    lse = jnp.log(jnp.sum(jnp.exp(shifted), axis=1, keepdims=True))
    out_ref[...] = shifted - lse


def _tc_stage_f(aggp, h1, degt, b, wl, bl):
    n = h1.shape[0]
    c = wl.shape[1]
    grid = n // _BR
    return pl.pallas_call(
        _tc_f_body,
        grid=(grid,),
        in_specs=[
            pl.BlockSpec((_NC, _BR, _H), lambda i: (0, i, 0)),
            pl.BlockSpec((_BR, _H), lambda i: (i, 0)),
            pl.BlockSpec((_BR, 2), lambda i: (i, 0)),
            pl.BlockSpec(b.shape, lambda i: (0, 0)),
            pl.BlockSpec(wl.shape, lambda i: (0, 0)),
            pl.BlockSpec(bl.shape, lambda i: (0, 0)),
        ],
        out_specs=pl.BlockSpec((_BR, c), lambda i: (i, 0)),
        out_shape=jax.ShapeDtypeStruct((n, c), jnp.float32),
    )(aggp, h1, degt, b, wl, bl)


@jax.jit
def kernel(x, edge_index, edge_attr, W0, b0, W1, b1, Wl, bl):
    src = edge_index[0]
    dst = edge_index[1]
    ew = edge_attr[:, 0]

    pad = _EPAD - _E
    src3 = jnp.pad(src, (0, pad)).reshape(_NW, _CPT, _CH)
    dst3 = jnp.pad(dst, (0, pad)).reshape(_NW, _CPT, _CH)
    ew3 = jnp.pad(ew, (0, pad)).reshape(_NW, _CPT, _CH)

    zrow = jnp.zeros((_NPAD,), jnp.float32)
    zblk = jnp.zeros((_CH, _H), jnp.float32)

    degp = _sc_deg()(dst3, ew3, zrow)
    degt = degp.T

    h0, hs0 = _tc_stage_b(x, W0, degt)
    agg1 = _sc_agg()(hs0, src3, dst3, ew3, zblk)
    h1, hs1 = _tc_stage_d(agg1, h0, degt, b0.reshape(1, _H), W1)
    agg2 = _sc_agg()(hs1, src3, dst3, ew3, zblk)
    out = _tc_stage_f(agg2, h1, degt, b1.reshape(1, _H), Wl, bl.reshape(1, -1))
    return out
